# baseline (speedup 1.0000x reference)
import functools

import jax
import jax.numpy as jnp
from jax import lax
from jax.experimental import pallas as pl
from jax.experimental.pallas import tpu as pltpu
from jax.experimental.pallas import tpu_sc as plsc

MAG = 0.5

_TOTAL = 16384 * 2048
_NW = 32
_PER_W = _TOTAL // _NW
_CH = 16384
_NCH = _PER_W // _CH
_VECS = _CH // 16


def _sc_map_chunk(src, dst):
    half = jnp.full((16,), MAG, jnp.float32)
    two = jnp.full((16,), 1.0 / MAG, jnp.float32)
    one = jnp.full((16,), 1.0, jnp.float32)

    @pl.loop(0, _VECS, unroll=8)
    def _vec(i):
        c = src[pl.ds(i * 16, 16)]
        dst[pl.ds(i * 16, 16)] = jnp.where(
            c == 1.0, half, jnp.where(c == -1.0, two, one))


def _sc_body(cond_hbm, out_hbm, in0, in1, ou0, ou1, sem_in, sem_out):
    wid = lax.axis_index("s") * 2 + lax.axis_index("c")
    base = wid * _PER_W
    inb = (in0, in1)
    oub = (ou0, ou1)

    pltpu.async_copy(cond_hbm.at[pl.ds(base, _CH)], in0, sem_in)
    pltpu.async_copy(cond_hbm.at[pl.ds(base + _CH, _CH)], in1, sem_in)

    @pl.loop(0, _NCH, step=2)
    def _pair(g0):
        for j in range(2):
            g = g0 + j
            src, dst = inb[j], oub[j]
            pltpu.make_async_copy(cond_hbm.at[pl.ds(0, _CH)], src, sem_in).wait()

            @pl.when(g >= 2)
            def _():
                pltpu.make_async_copy(
                    dst, out_hbm.at[pl.ds(0, _CH)], sem_out).wait()

            _sc_map_chunk(src, dst)
            pltpu.async_copy(dst, out_hbm.at[pl.ds(base + g * _CH, _CH)], sem_out)

            @pl.when(g + 2 < _NCH)
            def _():
                pltpu.async_copy(
                    cond_hbm.at[pl.ds(base + (g + 2) * _CH, _CH)], src, sem_in)

    pltpu.make_async_copy(ou0, out_hbm.at[pl.ds(0, _CH)], sem_out).wait()
    pltpu.make_async_copy(ou1, out_hbm.at[pl.ds(0, _CH)], sem_out).wait()


@jax.jit
def _sc_run(flat):
    mesh = plsc.VectorSubcoreMesh(core_axis_name="c", subcore_axis_name="s")
    return pl.kernel(
        _sc_body,
        out_type=jax.ShapeDtypeStruct((_TOTAL,), jnp.float32),
        mesh=mesh,
        scratch_types=[
            pltpu.VMEM((_CH,), jnp.float32),
            pltpu.VMEM((_CH,), jnp.float32),
            pltpu.VMEM((_CH,), jnp.float32),
            pltpu.VMEM((_CH,), jnp.float32),
            pltpu.SemaphoreType.DMA,
            pltpu.SemaphoreType.DMA,
        ],
    )(flat)


def kernel(cond):
    n, m = cond.shape
    return _sc_run(cond.reshape(-1)).reshape(n, m)


# speedup vs baseline: 1.7124x; 1.7124x over previous
import functools

import jax
import jax.numpy as jnp
from jax import lax
from jax.experimental import pallas as pl
from jax.experimental.pallas import tpu as pltpu
from jax.experimental.pallas import tpu_sc as plsc

MAG = 0.5

_TOTAL = 16384 * 2048
_NW = 32
_PER_W = _TOTAL // _NW
_CH = 16384
_NCH = _PER_W // _CH
_VECS = _CH // 16


_BATCH = 8


def _sc_map_chunk(src, dst):
    half = jnp.full((16,), MAG, jnp.float32)
    two = jnp.full((16,), 1.0 / MAG, jnp.float32)
    one = jnp.full((16,), 1.0, jnp.float32)

    @pl.loop(0, _VECS // _BATCH)
    def _vec(i):
        base = i * (16 * _BATCH)
        cs = [src[pl.ds(base + k * 16, 16)] for k in range(_BATCH)]
        rs = [jnp.where(c == 1.0, half, jnp.where(c == -1.0, two, one))
              for c in cs]
        for k in range(_BATCH):
            dst[pl.ds(base + k * 16, 16)] = rs[k]


def _sc_body(cond_hbm, out_hbm, in0, in1, ou0, ou1, sem_in, sem_out):
    wid = lax.axis_index("s") * 2 + lax.axis_index("c")
    base = wid * _PER_W
    inb = (in0, in1)
    oub = (ou0, ou1)

    pltpu.async_copy(cond_hbm.at[pl.ds(base, _CH)], in0, sem_in)
    pltpu.async_copy(cond_hbm.at[pl.ds(base + _CH, _CH)], in1, sem_in)

    @pl.loop(0, _NCH, step=2)
    def _pair(g0):
        for j in range(2):
            g = g0 + j
            src, dst = inb[j], oub[j]
            pltpu.make_async_copy(cond_hbm.at[pl.ds(0, _CH)], src, sem_in).wait()

            @pl.when(g >= 2)
            def _():
                pltpu.make_async_copy(
                    dst, out_hbm.at[pl.ds(0, _CH)], sem_out).wait()

            _sc_map_chunk(src, dst)
            pltpu.async_copy(dst, out_hbm.at[pl.ds(base + g * _CH, _CH)], sem_out)

            @pl.when(g + 2 < _NCH)
            def _():
                pltpu.async_copy(
                    cond_hbm.at[pl.ds(base + (g + 2) * _CH, _CH)], src, sem_in)

    pltpu.make_async_copy(ou0, out_hbm.at[pl.ds(0, _CH)], sem_out).wait()
    pltpu.make_async_copy(ou1, out_hbm.at[pl.ds(0, _CH)], sem_out).wait()


@jax.jit
def _sc_run(flat):
    mesh = plsc.VectorSubcoreMesh(core_axis_name="c", subcore_axis_name="s")
    return pl.kernel(
        _sc_body,
        out_type=jax.ShapeDtypeStruct((_TOTAL,), jnp.float32),
        mesh=mesh,
        scratch_types=[
            pltpu.VMEM((_CH,), jnp.float32),
            pltpu.VMEM((_CH,), jnp.float32),
            pltpu.VMEM((_CH,), jnp.float32),
            pltpu.VMEM((_CH,), jnp.float32),
            pltpu.SemaphoreType.DMA,
            pltpu.SemaphoreType.DMA,
        ],
    )(flat)


def kernel(cond):
    n, m = cond.shape
    return _sc_run(cond.reshape(-1)).reshape(n, m)
